# baseline (speedup 1.0000x reference)
import jax
import jax.numpy as jnp
from jax.experimental import pallas as pl

_N = 8192
_HALF = 4096
_D = 128
_MB = 512


def _pass1_kernel(e_ref, x_ref, wr_ref, wd_ref, dis_ref, y_ref):
    i = pl.program_id(0)
    s = jnp.sum(e_ref[...], axis=1, keepdims=True) + 1.0
    dis = jnp.where(s > 0.0, jax.lax.rsqrt(s), 0.0)
    dis_ref[...] = dis
    w = jnp.where(i * _MB < _HALF, wr_ref[...], wd_ref[...])
    y_ref[...] = dis * jnp.dot(x_ref[...], w, preferred_element_type=jnp.float32)


def _pass2_kernel(e_ref, y_ref, yi_ref, dis_ref, b_ref, o_ref):
    z = jnp.dot(e_ref[...], y_ref[...], preferred_element_type=jnp.float32)
    o = dis_ref[...] * (z + yi_ref[...]) + b_ref[...]
    o_ref[...] = jnp.where(o >= 0.0, o, 0.01 * o)


def kernel(x, edge_index, weightr, weightd, bias):
    nb = _N // _MB
    dis, y = pl.pallas_call(
        _pass1_kernel,
        grid=(nb,),
        in_specs=[
            pl.BlockSpec((_MB, _N), lambda i: (i, 0)),
            pl.BlockSpec((_MB, _D), lambda i: (i, 0)),
            pl.BlockSpec((_D, _D), lambda i: (0, 0)),
            pl.BlockSpec((_D, _D), lambda i: (0, 0)),
        ],
        out_specs=[
            pl.BlockSpec((_MB, 1), lambda i: (i, 0)),
            pl.BlockSpec((_MB, _D), lambda i: (i, 0)),
        ],
        out_shape=[
            jax.ShapeDtypeStruct((_N, 1), jnp.float32),
            jax.ShapeDtypeStruct((_N, _D), jnp.float32),
        ],
    )(edge_index, x, weightr, weightd)

    out = pl.pallas_call(
        _pass2_kernel,
        grid=(nb,),
        in_specs=[
            pl.BlockSpec((_MB, _N), lambda i: (i, 0)),
            pl.BlockSpec((_N, _D), lambda i: (0, 0)),
            pl.BlockSpec((_MB, _D), lambda i: (i, 0)),
            pl.BlockSpec((_MB, 1), lambda i: (i, 0)),
            pl.BlockSpec((1, _D), lambda i: (0, 0)),
        ],
        out_specs=pl.BlockSpec((_MB, _D), lambda i: (i, 0)),
        out_shape=jax.ShapeDtypeStruct((_N, _D), jnp.float32),
    )(edge_index, y, y, dis, bias.reshape(1, _D))
    return out


# speedup vs baseline: 1.0874x; 1.0874x over previous
import jax
import jax.numpy as jnp
from jax.experimental import pallas as pl
from jax.experimental.pallas import tpu as pltpu

_N = 8192
_HALF = 4096
_D = 128
_MB = 512
_NB = _N // _MB


def _gcn_kernel(e_ref, x_ref, wr_ref, wd_ref, b_ref, o_ref, y_scr, dis_scr):
    i = pl.program_id(0)

    @pl.when(i < _NB)
    def _phase1():
        s = jnp.sum(e_ref[...], axis=1, keepdims=True) + 1.0
        dis = jnp.where(s > 0.0, jax.lax.rsqrt(s), 0.0)
        dis_scr[pl.ds(i * _MB, _MB), :] = dis
        w = jnp.where(i * _MB < _HALF, wr_ref[...], wd_ref[...])
        y_scr[pl.ds(i * _MB, _MB), :] = dis * jnp.dot(
            x_ref[...], w, preferred_element_type=jnp.float32
        )

    @pl.when(i >= _NB)
    def _phase2():
        j = i - _NB
        z = jnp.dot(e_ref[...], y_scr[...], preferred_element_type=jnp.float32)
        o = (
            dis_scr[pl.ds(j * _MB, _MB), :] * (z + y_scr[pl.ds(j * _MB, _MB), :])
            + b_ref[...]
        )
        o_ref[...] = jnp.where(o >= 0.0, o, 0.01 * o)


def kernel(x, edge_index, weightr, weightd, bias):
    out = pl.pallas_call(
        _gcn_kernel,
        grid=(2 * _NB,),
        in_specs=[
            pl.BlockSpec((_MB, _N), lambda i: (i % _NB, 0)),
            pl.BlockSpec((_MB, _D), lambda i: (jnp.where(i < _NB, i, 0), 0)),
            pl.BlockSpec((_D, _D), lambda i: (0, 0)),
            pl.BlockSpec((_D, _D), lambda i: (0, 0)),
            pl.BlockSpec((1, _D), lambda i: (0, 0)),
        ],
        out_specs=pl.BlockSpec(
            (_MB, _D), lambda i: (jnp.where(i < _NB, 0, i - _NB), 0)
        ),
        out_shape=jax.ShapeDtypeStruct((_N, _D), jnp.float32),
        scratch_shapes=[
            pltpu.VMEM((_N, _D), jnp.float32),
            pltpu.VMEM((_N, 1), jnp.float32),
        ],
    )(edge_index, x, weightr, weightd, bias.reshape(1, _D))
    return out


# speedup vs baseline: 1.0950x; 1.0069x over previous
import jax
import jax.numpy as jnp
from jax.experimental import pallas as pl
from jax.experimental.pallas import tpu as pltpu

_N = 8192
_HALF = 4096
_D = 128
_MB = 256
_NB = _N // _MB


def _gcn_kernel(e_ref, x_ref, wr_ref, wd_ref, b_ref, o_ref, y_scr, dis_scr):
    i = pl.program_id(0)

    @pl.when(i < _NB)
    def _phase1():
        s = jnp.sum(e_ref[...], axis=1, keepdims=True) + 1.0
        dis = jnp.where(s > 0.0, jax.lax.rsqrt(s), 0.0)
        dis_scr[pl.ds(i * _MB, _MB), :] = dis
        w = jnp.where(i * _MB < _HALF, wr_ref[...], wd_ref[...])
        y_scr[pl.ds(i * _MB, _MB), :] = dis * jnp.dot(
            x_ref[...], w, preferred_element_type=jnp.float32
        )

    @pl.when(i >= _NB)
    def _phase2():
        j = i - _NB
        z = jnp.dot(e_ref[...], y_scr[...], preferred_element_type=jnp.float32)
        o = (
            dis_scr[pl.ds(j * _MB, _MB), :] * (z + y_scr[pl.ds(j * _MB, _MB), :])
            + b_ref[...]
        )
        o_ref[...] = jnp.where(o >= 0.0, o, 0.01 * o)


def kernel(x, edge_index, weightr, weightd, bias):
    out = pl.pallas_call(
        _gcn_kernel,
        grid=(2 * _NB,),
        in_specs=[
            pl.BlockSpec((_MB, _N), lambda i: (i % _NB, 0)),
            pl.BlockSpec((_MB, _D), lambda i: (jnp.where(i < _NB, i, 0), 0)),
            pl.BlockSpec((_D, _D), lambda i: (0, 0)),
            pl.BlockSpec((_D, _D), lambda i: (0, 0)),
            pl.BlockSpec((1, _D), lambda i: (0, 0)),
        ],
        out_specs=pl.BlockSpec(
            (_MB, _D), lambda i: (jnp.where(i < _NB, 0, i - _NB), 0)
        ),
        out_shape=jax.ShapeDtypeStruct((_N, _D), jnp.float32),
        scratch_shapes=[
            pltpu.VMEM((_N, _D), jnp.float32),
            pltpu.VMEM((_N, 1), jnp.float32),
        ],
    )(edge_index, x, weightr, weightd, bias.reshape(1, _D))
    return out
